# baseline (speedup 1.0000x reference)
import functools

import jax
import jax.numpy as jnp
from jax import lax
from jax.experimental import pallas as pl
from jax.experimental.pallas import tpu as pltpu
from jax.experimental.pallas import tpu_sc as plsc

_NV = 50000
_HD = 128
_NE = 400000
_NG = 64
_EPS = 1e-5

_NSUB = 16
_CH = 128
_NQ = 4
_QR = 12512
_ACR = 12544
_QDUMMY = 12512
_ZPS = 784
_NACC = _NQ * _QR
_NCHT = 3200
_PADT = _NCHT * _CH
_SLOTS = 256
_DC = _NCHT - 1
_BN = 1000
_NBLK = _NV // _BN


def _bucket_indices(ei):
    src = ei[0].astype(jnp.int32)
    dst = ei[1].astype(jnp.int32)
    q = dst // _QR
    order = jnp.argsort(q, stable=True)
    srcs = src[order]
    dsts = dst[order]
    qs = q[order]
    lens = jnp.bincount(q, length=_NQ)
    off = jnp.concatenate([jnp.zeros((1,), jnp.int32),
                           jnp.cumsum(lens).astype(jnp.int32)])
    grain = _NSUB * _CH
    plens = ((lens + grain - 1) // grain) * grain
    poff = jnp.concatenate([jnp.zeros((1,), jnp.int32),
                            jnp.cumsum(plens).astype(jnp.int32)])
    pos = poff[qs] + (jnp.arange(_NE, dtype=jnp.int32) - off[qs])
    src_pad = jnp.zeros((_PADT,), jnp.int32).at[pos].set(srcs)
    dstl_pad = jnp.full((_PADT,), _QDUMMY, jnp.int32).at[pos].set(
        dsts - qs * _QR)
    base = (poff[:_NQ] // _CH)[:, None, None]
    nch = (plens // _CH)[:, None, None]
    k = (jnp.arange(_SLOTS, dtype=jnp.int32)[None, None, :] * _NSUB
         + jnp.arange(_NSUB, dtype=jnp.int32)[None, :, None])
    table = jnp.where(k < nch, base + k, _DC).astype(jnp.int32)
    return (src_pad.reshape(_NCHT, _CH), dstl_pad.reshape(_NCHT, _CH),
            table.reshape(_NQ, _NSUB, 16, 16))


def _sc_mesh():
    return plsc.VectorSubcoreMesh(core_axis_name="c", subcore_axis_name="s",
                                  num_cores=2, num_subcores=_NSUB)


def _copy_out(acc, out, q, sidx):
    @pl.when(sidx < _NSUB - 1)
    def _():
        pltpu.sync_copy(
            acc.at[pl.ds(sidx * _ZPS, _ZPS)],
            out.at[pl.ds(q * _QR + sidx * _ZPS, _ZPS)])

    @pl.when(sidx == _NSUB - 1)
    def _():
        pltpu.sync_copy(
            acc.at[pl.ds(sidx * _ZPS, _QR - (_NSUB - 1) * _ZPS)],
            out.at[pl.ds(q * _QR + sidx * _ZPS,
                         _QR - (_NSUB - 1) * _ZPS)])


def _acc_round(q, s, ct16, sp, dp, z, out, acc,
               cidx16_v, src16_v, dst16_v, sem, chunk_body):
    pltpu.sync_copy(ct16.at[q, s], cidx16_v)
    pltpu.sync_copy(z, acc.at[pl.ds(s * _ZPS, _ZPS)])
    plsc.subcore_barrier()

    def make_body(lo, hi):
        def body(g, carry):
            vec = cidx16_v[g]
            pltpu.async_copy(sp.at[vec], src16_v, sem).wait()
            pltpu.async_copy(dp.at[vec], dst16_v, sem).wait()
            for k in range(lo, hi):
                @pl.when(vec[k] != _DC)
                def _(k=k):
                    chunk_body(src16_v, dst16_v, k)
            return carry
        return body

    lax.fori_loop(0, 16, make_body(0, 8), 0)
    lax.fori_loop(0, 16, make_body(8, 16), 0)
    plsc.subcore_barrier()
    _copy_out(acc, out, q, s)
    plsc.subcore_barrier()


def _sc_segsum(tab, srcp, dstp, ct16, z):
    @functools.partial(
        pl.kernel, mesh=_sc_mesh(),
        out_type=jax.ShapeDtypeStruct((_NACC, _HD), jnp.float32),
        scratch_types=[
            pltpu.VMEM((16, 16), jnp.int32),
            pltpu.VMEM((16, _CH), jnp.int32),
            pltpu.VMEM((16, _CH), jnp.int32),
            pltpu.VMEM((_CH, _HD), jnp.float32),
            pltpu.VMEM_SHARED((_ACR, _HD), jnp.float32),
            pltpu.SemaphoreType.DMA,
        ],
    )
    def run(tab_ref, sp, dp, ct16_ref, z_ref, out,
            cidx16_v, src16_v, dst16_v, rows_v, acc, sem):
        c = lax.axis_index("c")
        s = lax.axis_index("s")

        def chunk_body(src16, dst16, k):
            pltpu.async_copy(tab_ref.at[src16.at[k]], rows_v, sem).wait()
            pltpu.sync_copy(rows_v, acc.at[dst16.at[k]], add=True)

        for r in range(2):
            _acc_round(c * 2 + r, s, ct16_ref, sp, dp, z_ref, out,
                       acc, cidx16_v, src16_v, dst16_v, sem, chunk_body)

    return run(tab, srcp, dstp, ct16, z)


def _sc_counts(dstp, ct16, z, ones):
    @functools.partial(
        pl.kernel, mesh=_sc_mesh(),
        out_type=jax.ShapeDtypeStruct((_NACC, _HD), jnp.float32),
        scratch_types=[
            pltpu.VMEM((16, 16), jnp.int32),
            pltpu.VMEM((16, _CH), jnp.int32),
            pltpu.VMEM((16, _CH), jnp.int32),
            pltpu.VMEM((_CH, _HD), jnp.float32),
            pltpu.VMEM_SHARED((_ACR, _HD), jnp.float32),
            pltpu.SemaphoreType.DMA,
        ],
    )
    def run(dp, ct16_ref, z_ref, ones_ref, out,
            cidx16_v, src16_v, dst16_v, ones_v, acc, sem):
        c = lax.axis_index("c")
        s = lax.axis_index("s")
        pltpu.sync_copy(ones_ref, ones_v)

        def chunk_body(src16, dst16, k):
            pltpu.sync_copy(ones_v, acc.at[dst16.at[k]], add=True)

        for r in range(2):
            _acc_round(c * 2 + r, s, ct16_ref, dp, dp, z_ref, out,
                       acc, cidx16_v, src16_v, dst16_v, sem, chunk_body)

    return run(dstp, ct16, z, ones)


def _tc_embed(x, w, b):
    def kern(x_ref, w_ref, b_ref, o_ref):
        o_ref[...] = lax.dot_general(
            x_ref[...], w_ref[...], (((1,), (1,)), ((), ())),
            preferred_element_type=jnp.float32) + b_ref[...]

    return pl.pallas_call(
        kern,
        grid=(_NBLK,),
        in_specs=[
            pl.BlockSpec((_BN, _HD), lambda i: (i, 0)),
            pl.BlockSpec((_HD, _HD), lambda i: (0, 0)),
            pl.BlockSpec((1, _HD), lambda i: (0, 0)),
        ],
        out_specs=pl.BlockSpec((_BN, _HD), lambda i: (i, 0)),
        out_shape=jax.ShapeDtypeStruct((_NV, _HD), jnp.float32),
    )(x, w, b.reshape(1, _HD))


def _tc_combine(s1, c1, s2, c2, xd, wl1, wl2, wr1, wr2, bl1, bl2):
    def kern(s1r, c1r, s2r, c2r, x_ref, wl1r, wl2r, wr1r, wr2r, b1r, b2r,
             h_ref, st_ref):
        i = pl.program_id(0)
        agg1 = s1r[...] * (1.0 / jnp.maximum(c1r[...][:, :1], 1.0))
        agg2 = s2r[...] * (1.0 / jnp.maximum(c2r[...][:, :1], 1.0))
        wr = wr1r[...] + wr2r[...]
        dn = (((1,), (1,)), ((), ()))
        h = (lax.dot_general(agg1, wl1r[...], dn,
                             preferred_element_type=jnp.float32)
             + lax.dot_general(agg2, wl2r[...], dn,
                               preferred_element_type=jnp.float32)
             + lax.dot_general(x_ref[...], wr, dn,
                               preferred_element_type=jnp.float32)
             + b1r[...] + b2r[...]) * 0.5
        h_ref[...] = h

        @pl.when(i == 0)
        def _():
            st_ref[...] = jnp.zeros((8, _HD), jnp.float32)

        st_ref[...] += jnp.concatenate(
            [jnp.sum(h, axis=0, keepdims=True),
             jnp.sum(h * h, axis=0, keepdims=True),
             jnp.zeros((6, _HD), jnp.float32)], 0)

    big_spec = pl.BlockSpec((_BN, _HD), lambda i: (i, 0))
    w_spec = pl.BlockSpec((_HD, _HD), lambda i: (0, 0))
    b_spec = pl.BlockSpec((1, _HD), lambda i: (0, 0))
    return pl.pallas_call(
        kern,
        grid=(_NBLK,),
        in_specs=[big_spec] * 5 + [w_spec] * 4 + [b_spec] * 2,
        out_specs=[pl.BlockSpec((_BN, _HD), lambda i: (i, 0)),
                   pl.BlockSpec((8, _HD), lambda i: (0, 0))],
        out_shape=[jax.ShapeDtypeStruct((_NV, _HD), jnp.float32),
                   jax.ShapeDtypeStruct((8, _HD), jnp.float32)],
    )(s1, c1, s2, c2, xd, wl1, wl2, wr1, wr2,
      bl1.reshape(1, _HD), bl2.reshape(1, _HD))


def _tc_norm(h, st, gamma, beta, prev):
    has_prev = prev is not None

    def kern(*refs):
        if has_prev:
            h_ref, st_ref, g_ref, b_ref, p_ref, o_ref = refs
        else:
            h_ref, st_ref, g_ref, b_ref, o_ref = refs
        m = st_ref[0:1, :] * (1.0 / _NV)
        v = st_ref[1:2, :] * (1.0 / _NV) - m * m
        y = (h_ref[...] - m) * lax.rsqrt(v + _EPS) * g_ref[...] + b_ref[...]
        y = jnp.maximum(y, 0.0)
        if has_prev:
            y = y + p_ref[...]
        o_ref[...] = y

    in_specs = [pl.BlockSpec((_BN, _HD), lambda i: (i, 0)),
                pl.BlockSpec((8, _HD), lambda i: (0, 0)),
                pl.BlockSpec((1, _HD), lambda i: (0, 0)),
                pl.BlockSpec((1, _HD), lambda i: (0, 0))]
    args = [h, st, gamma.reshape(1, _HD), beta.reshape(1, _HD)]
    if has_prev:
        in_specs.append(pl.BlockSpec((_BN, _HD), lambda i: (i, 0)))
        args.append(prev)
    return pl.pallas_call(
        kern,
        grid=(_NBLK,),
        in_specs=in_specs,
        out_specs=pl.BlockSpec((_BN, _HD), lambda i: (i, 0)),
        out_shape=jax.ShapeDtypeStruct((_NV, _HD), jnp.float32),
    )(*args)


def _tc_pool_head(xv, bv, xp, bp, w_out, b_out):
    def kern(xv_ref, bvr, xp_ref, bpr, wo_ref, bo_ref, out_ref,
             sv, cv, sp_, cp_):
        i = pl.program_id(0)

        @pl.when(i == 0)
        def _():
            for r in (sv, cv, sp_, cp_):
                r[...] = jnp.zeros((_NG, _HD), jnp.float32)

        row_iota = lax.broadcasted_iota(jnp.int32, (_NG, _BN), 0)

        oh_v = (row_iota == bvr[0, 0, :][None, :]).astype(jnp.float32)
        sv[...] += jnp.dot(oh_v, xv_ref[...],
                           preferred_element_type=jnp.float32)
        cv[...] += jnp.broadcast_to(jnp.sum(oh_v, 1, keepdims=True),
                                    (_NG, _HD))

        oh_p = (row_iota == bpr[0, 0, :][None, :]).astype(jnp.float32)
        sp_[...] += jnp.dot(oh_p, xp_ref[...],
                            preferred_element_type=jnp.float32)
        cp_[...] += jnp.broadcast_to(jnp.sum(oh_p, 1, keepdims=True),
                                     (_NG, _HD))

        @pl.when(i == _NBLK - 1)
        def _():
            pv = sv[...] / jnp.maximum(cv[...], 1.0)
            pp = sp_[...] / jnp.maximum(cp_[...], 1.0)
            dn = (((1,), (1,)), ((), ()))
            z = (lax.dot_general(pv, wo_ref[0:1, :_HD], dn,
                                 preferred_element_type=jnp.float32)
                 + lax.dot_general(pp, wo_ref[0:1, _HD:], dn,
                                   preferred_element_type=jnp.float32)
                 + bo_ref[...])
            out_ref[...] = 1.0 / (1.0 + jnp.exp(-z))

    big_spec = pl.BlockSpec((_BN, _HD), lambda i: (i, 0))
    id_spec = pl.BlockSpec((1, 1, _BN), lambda i: (i, 0, 0))
    return pl.pallas_call(
        kern,
        grid=(_NBLK,),
        in_specs=[big_spec, id_spec, big_spec, id_spec,
                  pl.BlockSpec((1, 2 * _HD), lambda i: (0, 0)),
                  pl.BlockSpec((1, 1), lambda i: (0, 0))],
        out_specs=pl.BlockSpec((_NG, 1), lambda i: (0, 0)),
        out_shape=jax.ShapeDtypeStruct((_NG, 1), jnp.float32),
        scratch_shapes=[pltpu.VMEM((_NG, _HD), jnp.float32)] * 4,
    )(xv, bv, xp, bp, w_out, b_out.reshape(1, 1))


def kernel(x_vuln, x_patch, ei_vuln_calls_vuln, ei_vuln_aligned_patch, ei_patch_aligned_vuln, ei_patch_calls_patch, batch_vuln, W_emb_vuln, b_emb_vuln, batch_patch, W_emb_patch, b_emb_patch, Wl_0_vuln_calls_vuln, bl_0_vuln_calls_vuln, Wr_0_vuln_calls_vuln, Wl_0_vuln_aligned_patch, bl_0_vuln_aligned_patch, Wr_0_vuln_aligned_patch, Wl_0_patch_aligned_vuln, bl_0_patch_aligned_vuln, Wr_0_patch_aligned_vuln, Wl_0_patch_calls_patch, bl_0_patch_calls_patch, Wr_0_patch_calls_patch, gamma_0_vuln, beta_0_vuln, gamma_0_patch, beta_0_patch, Wl_1_vuln_calls_vuln, bl_1_vuln_calls_vuln, Wr_1_vuln_calls_vuln, Wl_1_vuln_aligned_patch, bl_1_vuln_aligned_patch, Wr_1_vuln_aligned_patch, Wl_1_patch_aligned_vuln, bl_1_patch_aligned_vuln, Wr_1_patch_aligned_vuln, Wl_1_patch_calls_patch, bl_1_patch_calls_patch, Wr_1_patch_calls_patch, gamma_1_vuln, beta_1_vuln, gamma_1_patch, beta_1_patch, W_out, b_out):
    z = jnp.zeros((_ZPS, _HD), jnp.float32)
    ones = jnp.ones((_CH, _HD), jnp.float32)

    idx = {
        "cv": _bucket_indices(ei_vuln_calls_vuln),
        "ap": _bucket_indices(ei_vuln_aligned_patch),
        "av": _bucket_indices(ei_patch_aligned_vuln),
        "cp": _bucket_indices(ei_patch_calls_patch),
    }
    counts = {et: _sc_counts(idx[et][1], idx[et][2], z, ones) for et in idx}

    xv = _tc_embed(x_vuln, W_emb_vuln, b_emb_vuln)
    xp = _tc_embed(x_patch, W_emb_patch, b_emb_patch)

    wl = {
        0: {"cv": Wl_0_vuln_calls_vuln, "ap": Wl_0_vuln_aligned_patch,
            "av": Wl_0_patch_aligned_vuln, "cp": Wl_0_patch_calls_patch},
        1: {"cv": Wl_1_vuln_calls_vuln, "ap": Wl_1_vuln_aligned_patch,
            "av": Wl_1_patch_aligned_vuln, "cp": Wl_1_patch_calls_patch},
    }
    bl = {
        0: {"cv": bl_0_vuln_calls_vuln, "ap": bl_0_vuln_aligned_patch,
            "av": bl_0_patch_aligned_vuln, "cp": bl_0_patch_calls_patch},
        1: {"cv": bl_1_vuln_calls_vuln, "ap": bl_1_vuln_aligned_patch,
            "av": bl_1_patch_aligned_vuln, "cp": bl_1_patch_calls_patch},
    }
    wr = {
        0: {"cv": Wr_0_vuln_calls_vuln, "ap": Wr_0_vuln_aligned_patch,
            "av": Wr_0_patch_aligned_vuln, "cp": Wr_0_patch_calls_patch},
        1: {"cv": Wr_1_vuln_calls_vuln, "ap": Wr_1_vuln_aligned_patch,
            "av": Wr_1_patch_aligned_vuln, "cp": Wr_1_patch_calls_patch},
    }
    gb = {
        0: {"v": (gamma_0_vuln, beta_0_vuln), "p": (gamma_0_patch, beta_0_patch)},
        1: {"v": (gamma_1_vuln, beta_1_vuln), "p": (gamma_1_patch, beta_1_patch)},
    }

    for i in range(2):
        s_cv = _sc_segsum(xv, *idx["cv"], z)
        s_ap = _sc_segsum(xv, *idx["ap"], z)
        s_av = _sc_segsum(xp, *idx["av"], z)
        s_cp = _sc_segsum(xp, *idx["cp"], z)

        h_v, st_v = _tc_combine(s_cv, counts["cv"], s_av, counts["av"], xv,
                                wl[i]["cv"], wl[i]["av"],
                                wr[i]["cv"], wr[i]["av"],
                                bl[i]["cv"], bl[i]["av"])
        h_p, st_p = _tc_combine(s_ap, counts["ap"], s_cp, counts["cp"], xp,
                                wl[i]["ap"], wl[i]["cp"],
                                wr[i]["ap"], wr[i]["cp"],
                                bl[i]["ap"], bl[i]["cp"])

        prev_v = xv if i > 0 else None
        prev_p = xp if i > 0 else None
        xv = _tc_norm(h_v, st_v, gb[i]["v"][0], gb[i]["v"][1], prev_v)
        xp = _tc_norm(h_p, st_p, gb[i]["p"][0], gb[i]["p"][1], prev_p)

    bv = batch_vuln.astype(jnp.int32).reshape(_NBLK, 1, _BN)
    bp = batch_patch.astype(jnp.int32).reshape(_NBLK, 1, _BN)
    return _tc_pool_head(xv, bv, xp, bp, W_out, b_out)


# speedup vs baseline: 1.4107x; 1.4107x over previous
import functools

import jax
import jax.numpy as jnp
from jax import lax
from jax.experimental import pallas as pl
from jax.experimental.pallas import tpu as pltpu
from jax.experimental.pallas import tpu_sc as plsc

_NV = 50000
_HD = 128
_NE = 400000
_NG = 64
_EPS = 1e-5

_NSUB = 16
_CH = 64
_NQ = 4
_QR = 12512
_ACR = 12544
_QDUMMY = 12512
_ZPS = 784
_NACC = _NQ * _QR
_GRAIN = 2048
_NPR = 3200
_PADT = _NPR * 2 * _CH
_BN = 1000
_NBLK = _NV // _BN


def _bucket_indices(ei):
    src = ei[0].astype(jnp.int32)
    dst = ei[1].astype(jnp.int32)
    q = dst // _QR
    order = jnp.argsort(q, stable=True)
    srcs = src[order]
    dsts = dst[order]
    qs = q[order]
    lens = jnp.bincount(q, length=_NQ)
    off = jnp.concatenate([jnp.zeros((1,), jnp.int32),
                           jnp.cumsum(lens).astype(jnp.int32)])
    plens = ((lens + _GRAIN - 1) // _GRAIN) * _GRAIN
    poff = jnp.concatenate([jnp.zeros((1,), jnp.int32),
                            jnp.cumsum(plens).astype(jnp.int32)])
    pos = poff[qs] + (jnp.arange(_NE, dtype=jnp.int32) - off[qs])
    src_pad = jnp.zeros((_PADT,), jnp.int32).at[pos].set(srcs)
    dstl_pad = jnp.full((_PADT,), _QDUMMY, jnp.int32).at[pos].set(
        dsts - qs * _QR)
    npairs = (plens // (2 * _CH * _NSUB))[:, None]
    start = ((poff[:_NQ] // (2 * _CH))[:, None]
             + jnp.arange(_NSUB, dtype=jnp.int32)[None, :] * npairs)
    meta = jnp.concatenate(
        [start[..., None], jnp.broadcast_to(npairs[..., None],
                                            (_NQ, _NSUB, 1)),
         jnp.zeros((_NQ, _NSUB, 14), jnp.int32)], -1).astype(jnp.int32)
    return (src_pad.reshape(_NPR, 2, _CH), dstl_pad.reshape(_NPR, 2, _CH),
            meta.reshape(_NQ, _NSUB, 1, 16))


def _sc_mesh():
    return plsc.VectorSubcoreMesh(core_axis_name="c", subcore_axis_name="s",
                                  num_cores=2, num_subcores=_NSUB)


def _copy_out(acc, out, q, sidx):
    @pl.when(sidx < _NSUB - 1)
    def _():
        pltpu.sync_copy(
            acc.at[pl.ds(sidx * _ZPS, _ZPS)],
            out.at[pl.ds(q * _QR + sidx * _ZPS, _ZPS)])

    @pl.when(sidx == _NSUB - 1)
    def _():
        pltpu.sync_copy(
            acc.at[pl.ds(sidx * _ZPS, _QR - (_NSUB - 1) * _ZPS)],
            out.at[pl.ds(q * _QR + sidx * _ZPS,
                         _QR - (_NSUB - 1) * _ZPS)])


def _acc_round(q, s, meta, z, out, acc, meta_v, pair_body):
    pltpu.sync_copy(meta.at[q, s], meta_v)
    pltpu.sync_copy(z, acc.at[pl.ds(s * _ZPS, _ZPS)])
    vecm = meta_v[0]
    start = vecm[0]
    npairs = vecm[1]
    plsc.subcore_barrier()

    def body(j, carry):
        pair_body(start + j)
        return carry

    lax.fori_loop(0, npairs, body, 0)
    plsc.subcore_barrier()
    _copy_out(acc, out, q, s)
    plsc.subcore_barrier()


def _sc_segsum(tab, srcp, dstp, meta, z):
    @functools.partial(
        pl.kernel, mesh=_sc_mesh(),
        out_type=jax.ShapeDtypeStruct((_NACC, _HD), jnp.float32),
        scratch_types=[
            pltpu.VMEM((1, 16), jnp.int32),
            pltpu.VMEM((2, _CH), jnp.int32),
            pltpu.VMEM((2, _CH), jnp.int32),
            pltpu.VMEM((_CH, _HD), jnp.float32),
            pltpu.VMEM((_CH, _HD), jnp.float32),
            pltpu.VMEM_SHARED((_ACR, _HD), jnp.float32),
            pltpu.SemaphoreType.DMA,
            pltpu.SemaphoreType.DMA,
        ],
    )
    def run(tab_ref, sp, dp, meta_ref, z_ref, out,
            meta_v, src_v, dst_v, rows_a, rows_b, acc, sem_a, sem_b):
        c = lax.axis_index("c")
        s = lax.axis_index("s")

        def pair_body(p):
            pltpu.sync_copy(sp.at[p], src_v)
            pltpu.sync_copy(dp.at[p], dst_v)
            ga = pltpu.async_copy(tab_ref.at[src_v.at[0]], rows_a, sem_a)
            gb = pltpu.async_copy(tab_ref.at[src_v.at[1]], rows_b, sem_b)
            ga.wait()
            pltpu.sync_copy(rows_a, acc.at[dst_v.at[0]], add=True)
            gb.wait()
            pltpu.sync_copy(rows_b, acc.at[dst_v.at[1]], add=True)

        for r in range(2):
            _acc_round(c * 2 + r, s, meta_ref, z_ref, out, acc, meta_v,
                       pair_body)

    return run(tab, srcp, dstp, meta, z)


def _sc_counts(dstp, meta, z, ones):
    @functools.partial(
        pl.kernel, mesh=_sc_mesh(),
        out_type=jax.ShapeDtypeStruct((_NACC, _HD), jnp.float32),
        scratch_types=[
            pltpu.VMEM((1, 16), jnp.int32),
            pltpu.VMEM((2, _CH), jnp.int32),
            pltpu.VMEM((_CH, _HD), jnp.float32),
            pltpu.VMEM_SHARED((_ACR, _HD), jnp.float32),
        ],
    )
    def run(dp, meta_ref, z_ref, ones_ref, out, meta_v, dst_v, ones_v, acc):
        c = lax.axis_index("c")
        s = lax.axis_index("s")
        pltpu.sync_copy(ones_ref, ones_v)

        def pair_body(p):
            pltpu.sync_copy(dp.at[p], dst_v)
            pltpu.sync_copy(ones_v, acc.at[dst_v.at[0]], add=True)
            pltpu.sync_copy(ones_v, acc.at[dst_v.at[1]], add=True)

        for r in range(2):
            _acc_round(c * 2 + r, s, meta_ref, z_ref, out, acc, meta_v,
                       pair_body)

    return run(dstp, meta, z, ones)


def _tc_embed(x, w, b):
    def kern(x_ref, w_ref, b_ref, o_ref):
        o_ref[...] = lax.dot_general(
            x_ref[...], w_ref[...], (((1,), (1,)), ((), ())),
            preferred_element_type=jnp.float32) + b_ref[...]

    return pl.pallas_call(
        kern,
        grid=(_NBLK,),
        in_specs=[
            pl.BlockSpec((_BN, _HD), lambda i: (i, 0)),
            pl.BlockSpec((_HD, _HD), lambda i: (0, 0)),
            pl.BlockSpec((1, _HD), lambda i: (0, 0)),
        ],
        out_specs=pl.BlockSpec((_BN, _HD), lambda i: (i, 0)),
        out_shape=jax.ShapeDtypeStruct((_NV, _HD), jnp.float32),
    )(x, w, b.reshape(1, _HD))


def _tc_combine(s1, c1, s2, c2, xd, wl1, wl2, wr1, wr2, bl1, bl2):
    def kern(s1r, c1r, s2r, c2r, x_ref, wl1r, wl2r, wr1r, wr2r, b1r, b2r,
             h_ref, st_ref):
        i = pl.program_id(0)
        agg1 = s1r[...] * (1.0 / jnp.maximum(c1r[...][:, :1], 1.0))
        agg2 = s2r[...] * (1.0 / jnp.maximum(c2r[...][:, :1], 1.0))
        wr = wr1r[...] + wr2r[...]
        dn = (((1,), (1,)), ((), ()))
        h = (lax.dot_general(agg1, wl1r[...], dn,
                             preferred_element_type=jnp.float32)
             + lax.dot_general(agg2, wl2r[...], dn,
                               preferred_element_type=jnp.float32)
             + lax.dot_general(x_ref[...], wr, dn,
                               preferred_element_type=jnp.float32)
             + b1r[...] + b2r[...]) * 0.5
        h_ref[...] = h

        @pl.when(i == 0)
        def _():
            st_ref[...] = jnp.zeros((8, _HD), jnp.float32)

        st_ref[...] += jnp.concatenate(
            [jnp.sum(h, axis=0, keepdims=True),
             jnp.sum(h * h, axis=0, keepdims=True),
             jnp.zeros((6, _HD), jnp.float32)], 0)

    big_spec = pl.BlockSpec((_BN, _HD), lambda i: (i, 0))
    w_spec = pl.BlockSpec((_HD, _HD), lambda i: (0, 0))
    b_spec = pl.BlockSpec((1, _HD), lambda i: (0, 0))
    return pl.pallas_call(
        kern,
        grid=(_NBLK,),
        in_specs=[big_spec] * 5 + [w_spec] * 4 + [b_spec] * 2,
        out_specs=[pl.BlockSpec((_BN, _HD), lambda i: (i, 0)),
                   pl.BlockSpec((8, _HD), lambda i: (0, 0))],
        out_shape=[jax.ShapeDtypeStruct((_NV, _HD), jnp.float32),
                   jax.ShapeDtypeStruct((8, _HD), jnp.float32)],
    )(s1, c1, s2, c2, xd, wl1, wl2, wr1, wr2,
      bl1.reshape(1, _HD), bl2.reshape(1, _HD))


def _tc_norm(h, st, gamma, beta, prev):
    has_prev = prev is not None

    def kern(*refs):
        if has_prev:
            h_ref, st_ref, g_ref, b_ref, p_ref, o_ref = refs
        else:
            h_ref, st_ref, g_ref, b_ref, o_ref = refs
        m = st_ref[0:1, :] * (1.0 / _NV)
        v = st_ref[1:2, :] * (1.0 / _NV) - m * m
        y = (h_ref[...] - m) * lax.rsqrt(v + _EPS) * g_ref[...] + b_ref[...]
        y = jnp.maximum(y, 0.0)
        if has_prev:
            y = y + p_ref[...]
        o_ref[...] = y

    in_specs = [pl.BlockSpec((_BN, _HD), lambda i: (i, 0)),
                pl.BlockSpec((8, _HD), lambda i: (0, 0)),
                pl.BlockSpec((1, _HD), lambda i: (0, 0)),
                pl.BlockSpec((1, _HD), lambda i: (0, 0))]
    args = [h, st, gamma.reshape(1, _HD), beta.reshape(1, _HD)]
    if has_prev:
        in_specs.append(pl.BlockSpec((_BN, _HD), lambda i: (i, 0)))
        args.append(prev)
    return pl.pallas_call(
        kern,
        grid=(_NBLK,),
        in_specs=in_specs,
        out_specs=pl.BlockSpec((_BN, _HD), lambda i: (i, 0)),
        out_shape=jax.ShapeDtypeStruct((_NV, _HD), jnp.float32),
    )(*args)


def _tc_pool_head(xv, bv, xp, bp, w_out, b_out):
    def kern(xv_ref, bvr, xp_ref, bpr, wo_ref, bo_ref, out_ref,
             sv, cv, sp_, cp_):
        i = pl.program_id(0)

        @pl.when(i == 0)
        def _():
            for r in (sv, cv, sp_, cp_):
                r[...] = jnp.zeros((_NG, _HD), jnp.float32)

        row_iota = lax.broadcasted_iota(jnp.int32, (_NG, _BN), 0)

        oh_v = (row_iota == bvr[0, 0, :][None, :]).astype(jnp.float32)
        sv[...] += jnp.dot(oh_v, xv_ref[...],
                           preferred_element_type=jnp.float32)
        cv[...] += jnp.broadcast_to(jnp.sum(oh_v, 1, keepdims=True),
                                    (_NG, _HD))

        oh_p = (row_iota == bpr[0, 0, :][None, :]).astype(jnp.float32)
        sp_[...] += jnp.dot(oh_p, xp_ref[...],
                            preferred_element_type=jnp.float32)
        cp_[...] += jnp.broadcast_to(jnp.sum(oh_p, 1, keepdims=True),
                                     (_NG, _HD))

        @pl.when(i == _NBLK - 1)
        def _():
            pv = sv[...] / jnp.maximum(cv[...], 1.0)
            pp = sp_[...] / jnp.maximum(cp_[...], 1.0)
            dn = (((1,), (1,)), ((), ()))
            z = (lax.dot_general(pv, wo_ref[0:1, :_HD], dn,
                                 preferred_element_type=jnp.float32)
                 + lax.dot_general(pp, wo_ref[0:1, _HD:], dn,
                                   preferred_element_type=jnp.float32)
                 + bo_ref[...])
            out_ref[...] = 1.0 / (1.0 + jnp.exp(-z))

    big_spec = pl.BlockSpec((_BN, _HD), lambda i: (i, 0))
    id_spec = pl.BlockSpec((1, 1, _BN), lambda i: (i, 0, 0))
    return pl.pallas_call(
        kern,
        grid=(_NBLK,),
        in_specs=[big_spec, id_spec, big_spec, id_spec,
                  pl.BlockSpec((1, 2 * _HD), lambda i: (0, 0)),
                  pl.BlockSpec((1, 1), lambda i: (0, 0))],
        out_specs=pl.BlockSpec((_NG, 1), lambda i: (0, 0)),
        out_shape=jax.ShapeDtypeStruct((_NG, 1), jnp.float32),
        scratch_shapes=[pltpu.VMEM((_NG, _HD), jnp.float32)] * 4,
    )(xv, bv, xp, bp, w_out, b_out.reshape(1, 1))


def kernel(x_vuln, x_patch, ei_vuln_calls_vuln, ei_vuln_aligned_patch, ei_patch_aligned_vuln, ei_patch_calls_patch, batch_vuln, W_emb_vuln, b_emb_vuln, batch_patch, W_emb_patch, b_emb_patch, Wl_0_vuln_calls_vuln, bl_0_vuln_calls_vuln, Wr_0_vuln_calls_vuln, Wl_0_vuln_aligned_patch, bl_0_vuln_aligned_patch, Wr_0_vuln_aligned_patch, Wl_0_patch_aligned_vuln, bl_0_patch_aligned_vuln, Wr_0_patch_aligned_vuln, Wl_0_patch_calls_patch, bl_0_patch_calls_patch, Wr_0_patch_calls_patch, gamma_0_vuln, beta_0_vuln, gamma_0_patch, beta_0_patch, Wl_1_vuln_calls_vuln, bl_1_vuln_calls_vuln, Wr_1_vuln_calls_vuln, Wl_1_vuln_aligned_patch, bl_1_vuln_aligned_patch, Wr_1_vuln_aligned_patch, Wl_1_patch_aligned_vuln, bl_1_patch_aligned_vuln, Wr_1_patch_aligned_vuln, Wl_1_patch_calls_patch, bl_1_patch_calls_patch, Wr_1_patch_calls_patch, gamma_1_vuln, beta_1_vuln, gamma_1_patch, beta_1_patch, W_out, b_out):
    z = jnp.zeros((_ZPS, _HD), jnp.float32)
    ones = jnp.ones((_CH, _HD), jnp.float32)

    idx = {
        "cv": _bucket_indices(ei_vuln_calls_vuln),
        "ap": _bucket_indices(ei_vuln_aligned_patch),
        "av": _bucket_indices(ei_patch_aligned_vuln),
        "cp": _bucket_indices(ei_patch_calls_patch),
    }
    counts = {et: _sc_counts(idx[et][1], idx[et][2], z, ones) for et in idx}

    xv = _tc_embed(x_vuln, W_emb_vuln, b_emb_vuln)
    xp = _tc_embed(x_patch, W_emb_patch, b_emb_patch)

    wl = {
        0: {"cv": Wl_0_vuln_calls_vuln, "ap": Wl_0_vuln_aligned_patch,
            "av": Wl_0_patch_aligned_vuln, "cp": Wl_0_patch_calls_patch},
        1: {"cv": Wl_1_vuln_calls_vuln, "ap": Wl_1_vuln_aligned_patch,
            "av": Wl_1_patch_aligned_vuln, "cp": Wl_1_patch_calls_patch},
    }
    bl = {
        0: {"cv": bl_0_vuln_calls_vuln, "ap": bl_0_vuln_aligned_patch,
            "av": bl_0_patch_aligned_vuln, "cp": bl_0_patch_calls_patch},
        1: {"cv": bl_1_vuln_calls_vuln, "ap": bl_1_vuln_aligned_patch,
            "av": bl_1_patch_aligned_vuln, "cp": bl_1_patch_calls_patch},
    }
    wr = {
        0: {"cv": Wr_0_vuln_calls_vuln, "ap": Wr_0_vuln_aligned_patch,
            "av": Wr_0_patch_aligned_vuln, "cp": Wr_0_patch_calls_patch},
        1: {"cv": Wr_1_vuln_calls_vuln, "ap": Wr_1_vuln_aligned_patch,
            "av": Wr_1_patch_aligned_vuln, "cp": Wr_1_patch_calls_patch},
    }
    gb = {
        0: {"v": (gamma_0_vuln, beta_0_vuln), "p": (gamma_0_patch, beta_0_patch)},
        1: {"v": (gamma_1_vuln, beta_1_vuln), "p": (gamma_1_patch, beta_1_patch)},
    }

    for i in range(2):
        s_cv = _sc_segsum(xv, *idx["cv"], z)
        s_ap = _sc_segsum(xv, *idx["ap"], z)
        s_av = _sc_segsum(xp, *idx["av"], z)
        s_cp = _sc_segsum(xp, *idx["cp"], z)

        h_v, st_v = _tc_combine(s_cv, counts["cv"], s_av, counts["av"], xv,
                                wl[i]["cv"], wl[i]["av"],
                                wr[i]["cv"], wr[i]["av"],
                                bl[i]["cv"], bl[i]["av"])
        h_p, st_p = _tc_combine(s_ap, counts["ap"], s_cp, counts["cp"], xp,
                                wl[i]["ap"], wl[i]["cp"],
                                wr[i]["ap"], wr[i]["cp"],
                                bl[i]["ap"], bl[i]["cp"])

        prev_v = xv if i > 0 else None
        prev_p = xp if i > 0 else None
        xv = _tc_norm(h_v, st_v, gb[i]["v"][0], gb[i]["v"][1], prev_v)
        xp = _tc_norm(h_p, st_p, gb[i]["p"][0], gb[i]["p"][1], prev_p)

    bv = batch_vuln.astype(jnp.int32).reshape(_NBLK, 1, _BN)
    bp = batch_patch.astype(jnp.int32).reshape(_NBLK, 1, _BN)
    return _tc_pool_head(xv, bv, xp, bp, W_out, b_out)
